# baseline (speedup 1.0000x reference)
import functools

import jax
import jax.numpy as jnp
from jax import lax
from jax.experimental import pallas as pl
from jax.experimental.pallas import tpu as pltpu
from jax.experimental.pallas import tpu_sc as plsc

N, E, G, H = 10000, 320000, 64, 128
ROWC = 80
NROWC = N // ROWC
NTPC = 16
NW = 32
f32 = jnp.float32
i32 = jnp.int32

CH1 = 512
KR1 = CH1 // 128
NC1 = E // CH1
IT1 = 40
CH2 = 128
NC2 = E // CH2
IT2 = 80
CH4 = 128
NC4 = E // CH4
IT4 = 80


def _mesh():
    return plsc.VectorSubcoreMesh(core_axis_name="c", subcore_axis_name="s")


EBLK = 1000


def _t01_body(aref, oref, we, be_r, weo, beo_r, wl1, bl1r, wl2, bl2r,
              wl3, bl3r, ea1_o, q2_o, q3_o):
    a = aref[...]
    o = oref[...]
    eel = a * we[...] + be_r[...]
    eer = o * weo[...] + beo_r[...]
    ee = jnp.concatenate([eel, eer], axis=1)
    ea1_o[...] = jnp.dot(ee, wl1[...], preferred_element_type=f32) + bl1r[...]
    q2_o[...] = jnp.dot(ee, wl2[...], preferred_element_type=f32) + bl2r[...]
    q3_o[...] = jnp.dot(ee, wl3[...], preferred_element_type=f32) + bl3r[...]


def _t01(a2, o2, We, be, Weo, beo, Wl1, bl1, Wl2, bl2, Wl3, bl3):
    nb = E // EBLK
    return pl.pallas_call(
        _t01_body,
        grid=(nb,),
        in_specs=[
            pl.BlockSpec((EBLK, 1), lambda i: (i, 0)),
            pl.BlockSpec((EBLK, 1), lambda i: (i, 0)),
            pl.BlockSpec((1, 64), lambda i: (0, 0)),
            pl.BlockSpec((1, 64), lambda i: (0, 0)),
            pl.BlockSpec((1, 64), lambda i: (0, 0)),
            pl.BlockSpec((1, 64), lambda i: (0, 0)),
            pl.BlockSpec((H, 1), lambda i: (0, 0)),
            pl.BlockSpec((1, 1), lambda i: (0, 0)),
            pl.BlockSpec((H, H), lambda i: (0, 0)),
            pl.BlockSpec((1, H), lambda i: (0, 0)),
            pl.BlockSpec((H, H), lambda i: (0, 0)),
            pl.BlockSpec((1, H), lambda i: (0, 0)),
        ],
        out_specs=[
            pl.BlockSpec((EBLK, 1), lambda i: (i, 0)),
            pl.BlockSpec((EBLK, H), lambda i: (i, 0)),
            pl.BlockSpec((EBLK, H), lambda i: (i, 0)),
        ],
        out_shape=[
            jax.ShapeDtypeStruct((E, 1), f32),
            jax.ShapeDtypeStruct((E, H), f32),
            jax.ShapeDtypeStruct((E, H), f32),
        ],
    )(a2, o2, We, be.reshape(1, 64), Weo, beo.reshape(1, 64),
      Wl1, bl1.reshape(1, 1), Wl2, bl2.reshape(1, H), Wl3, bl3.reshape(1, H))


def _k1_body(srcr, dstr, ea1, xf, s_out,
             rs0, rs1, rd0, rd1, fb0, fb1, g0, g1, mbuf, aggr, sem0, sem1):
    cid = lax.axis_index("c")
    sid = lax.axis_index("s")
    rs = (rs0, rs1)
    rd = (rd0, rd1)
    fb = (fb0, fb1)
    g = (g0, g1)
    sem = (sem0, sem1)

    @pl.when(cid == 0)
    def _core0():
        def initb(i, carry):
            c = i * NTPC + sid

            @pl.when(c < NROWC)
            def _():
                pltpu.sync_copy(xf.at[pl.ds(c * ROWC, ROWC)],
                                mbuf.at[pl.ds(0, ROWC)])
                pltpu.sync_copy(mbuf.at[pl.ds(0, ROWC)],
                                aggr.at[pl.ds(c * ROWC, ROWC)])
            return carry

        lax.fori_loop(0, (NROWC + NTPC - 1) // NTPC, initb, 0)
        plsc.subcore_barrier()

        def _base(k):
            c = k * NTPC + sid
            cc = jnp.minimum(c, NC1 - 1)
            return cc * CH1, cc * KR1, c

        def _gather(q):
            for j in range(KR1):
                pltpu.async_copy(xf.at[rs[q].at[j]],
                                 g[q].at[pl.ds(j * 128, 128)], sem[q])

        base0, rbase0, _ = _base(0)
        pltpu.sync_copy(srcr.at[pl.ds(rbase0, KR1)], rs[0])
        pltpu.sync_copy(dstr.at[pl.ds(rbase0, KR1)], rd[0])
        pltpu.sync_copy(ea1.at[pl.ds(base0, CH1)], fb[0])
        _gather(0)

        def step(i, carry):
            for b in (0, 1):
                k = i * 2 + b
                q = 1 - b
                _, _, c = _base(k)
                basen, rbasen, _ = _base(k + 1)
                pltpu.sync_copy(srcr.at[pl.ds(rbasen, KR1)], rs[q])
                pltpu.sync_copy(dstr.at[pl.ds(rbasen, KR1)], rd[q])
                pltpu.sync_copy(ea1.at[pl.ds(basen, CH1)], fb[q])
                _gather(q)
                pltpu.make_async_copy(xf.at[pl.ds(0, CH1)], g[b],
                                      sem[b]).wait()
                for gg in range(CH1 // 16):
                    sl = pl.ds(gg * 16, 16)
                    mbuf[sl] = jnp.maximum(g[b][sl] + fb[b][sl], 0.0)

                @pl.when(c < NC1)
                def _():
                    for j in range(KR1):
                        pltpu.sync_copy(mbuf.at[pl.ds(j * 128, 128)],
                                        aggr.at[rd[b].at[j]], add=True)
            return carry

        lax.fori_loop(0, IT1 // 2, step, 0)
        pltpu.make_async_copy(xf.at[pl.ds(0, CH1)], g[0], sem[0]).wait()
        plsc.subcore_barrier()

        def wb(i, carry):
            c = i * NTPC + sid

            @pl.when(c < NROWC)
            def _():
                pltpu.sync_copy(aggr.at[pl.ds(c * ROWC, ROWC)],
                                mbuf.at[pl.ds(0, ROWC)])
                pltpu.sync_copy(mbuf.at[pl.ds(0, ROWC)],
                                s_out.at[pl.ds(c * ROWC, ROWC)])
            return carry

        lax.fori_loop(0, (NROWC + NTPC - 1) // NTPC, wb, 0)


def _k1(srcr, dstr, ea1, xf):
    return pl.kernel(
        _k1_body,
        out_type=jax.ShapeDtypeStruct((N,), f32),
        mesh=_mesh(),
        scratch_types=[
            pltpu.VMEM((KR1, 128), i32),
            pltpu.VMEM((KR1, 128), i32),
            pltpu.VMEM((KR1, 128), i32),
            pltpu.VMEM((KR1, 128), i32),
            pltpu.VMEM((CH1,), f32),
            pltpu.VMEM((CH1,), f32),
            pltpu.VMEM((CH1,), f32),
            pltpu.VMEM((CH1,), f32),
            pltpu.VMEM((CH1,), f32),
            pltpu.VMEM_SHARED((N,), f32),
            pltpu.SemaphoreType.DMA,
            pltpu.SemaphoreType.DMA,
        ],
    )(srcr, dstr, ea1, xf)


def _k2_body(srcr, dstr, q2, s_hbm, coef, p2,
             rs0, rs1, rd0, rd1, qb0, qb1, g0, g1, cbuf, aggr, sem0, sem1):
    cid = lax.axis_index("c")
    sid = lax.axis_index("s")
    wid = cid * NTPC + sid
    rs = (rs0, rs1)
    rd = (rd0, rd1)
    qb = (qb0, qb1)
    g = (g0, g1)
    sem = (sem0, sem1)

    pltpu.sync_copy(coef, cbuf)
    w1s = [cbuf[0, pl.ds(gg * 16, 16)] for gg in range(8)]
    bns = [cbuf[1, pl.ds(gg * 16, 16)] for gg in range(8)]

    zv = jnp.zeros((16,), f32)

    def zb(i, carry):
        for gg in range(8):
            qb0[i, pl.ds(gg * 16, 16)] = zv
        return carry

    lax.fori_loop(0, CH2, zb, 0)

    def zc(i, carry):
        c = i * NTPC + sid

        @pl.when(c < NROWC)
        def _():
            pltpu.sync_copy(qb0.at[pl.ds(0, ROWC)],
                            aggr.at[pl.ds(c * ROWC, ROWC)])
        return carry

    lax.fori_loop(0, (NROWC + NTPC - 1) // NTPC, zc, 0)
    plsc.subcore_barrier()

    def _base(k):
        c = k * NW + wid
        cc = jnp.minimum(c, NC2 - 1)
        return cc * CH2, cc, c

    def _fetch(q, base, rbase):
        pltpu.sync_copy(srcr.at[pl.ds(rbase, 1)], rs[q])
        pltpu.sync_copy(dstr.at[pl.ds(rbase, 1)], rd[q])
        pltpu.async_copy(q2.at[pl.ds(base, CH2)], qb[q], sem[q])
        pltpu.async_copy(s_hbm.at[rs[q].at[0]], g[q], sem[q])

    base0, rbase0, _ = _base(0)
    _fetch(0, base0, rbase0)

    def step(i, carry):
        for b in (0, 1):
            k = i * 2 + b
            q = 1 - b
            _, _, c = _base(k)
            basen, rbasen, _ = _base(k + 1)
            _fetch(q, basen, rbasen)
            pltpu.make_async_copy(q2.at[pl.ds(0, CH2)], qb[b], sem[b]).wait()
            pltpu.make_async_copy(s_hbm.at[pl.ds(0, CH2)], g[b],
                                  sem[b]).wait()

            def edgegrp(qq, carry2):
                qbase = qq * 16
                sv = g[b][pl.ds(qbase, 16)]
                for e in range(16):
                    se = sv[e]
                    for gg in range(8):
                        sl = pl.ds(gg * 16, 16)
                        qb[b][qbase + e, sl] = jnp.maximum(
                            (se * w1s[gg] + bns[gg]) + qb[b][qbase + e, sl],
                            0.0)
                return carry2

            lax.fori_loop(0, CH2 // 16, edgegrp, 0)

            @pl.when(c < NC2)
            def _():
                pltpu.sync_copy(qb[b], aggr.at[rd[b].at[0]], add=True)
        return carry

    lax.fori_loop(0, IT2 // 2, step, 0)
    pltpu.make_async_copy(q2.at[pl.ds(0, CH2)], qb[0], sem[0]).wait()
    pltpu.make_async_copy(s_hbm.at[pl.ds(0, CH2)], g[0], sem[0]).wait()
    plsc.subcore_barrier()

    def wb(i, carry):
        c = i * NTPC + sid

        @pl.when(c < NROWC)
        def _():
            base = c * ROWC
            msl = qb0.at[pl.ds(0, ROWC)]
            pltpu.sync_copy(aggr.at[pl.ds(base, ROWC)], msl)

            @pl.when(cid == 0)
            def _():
                pltpu.sync_copy(s_hbm.at[pl.ds(base, ROWC)],
                                g0.at[pl.ds(0, ROWC)])

                def rowgrp(qq, carry2):
                    qbase = qq * 16
                    sv = g0[pl.ds(qbase, 16)]
                    for e in range(16):
                        se = sv[e]
                        for gg in range(8):
                            sl = pl.ds(gg * 16, 16)
                            qb0[qbase + e, sl] = qb0[qbase + e, sl] \
                                + se * w1s[gg] + bns[gg]
                    return carry2

                lax.fori_loop(0, ROWC // 16, rowgrp, 0)
            pltpu.sync_copy(msl, p2.at[pl.ds(cid * N + base, ROWC)])
        return carry

    lax.fori_loop(0, (NROWC + NTPC - 1) // NTPC, wb, 0)


def _k2(srcr, dstr, q2, s, coef):
    return pl.kernel(
        _k2_body,
        out_type=jax.ShapeDtypeStruct((2 * N, H), f32),
        mesh=_mesh(),
        scratch_types=[
            pltpu.VMEM((1, 128), i32),
            pltpu.VMEM((1, 128), i32),
            pltpu.VMEM((1, 128), i32),
            pltpu.VMEM((1, 128), i32),
            pltpu.VMEM((CH2, H), f32),
            pltpu.VMEM((CH2, H), f32),
            pltpu.VMEM((CH2,), f32),
            pltpu.VMEM((CH2,), f32),
            pltpu.VMEM((2, H), f32),
            pltpu.VMEM_SHARED((N, H), f32),
            pltpu.SemaphoreType.DMA,
            pltpu.SemaphoreType.DMA,
        ],
    )(srcr, dstr, q2, s, coef)


def _k4_body(srcr, dstr, q3, h2_hbm, p3,
             rs0, rs1, rd0, rd1, qbuf, g0, g1, aggr, sem0, sem1):
    cid = lax.axis_index("c")
    sid = lax.axis_index("s")
    wid = cid * NTPC + sid
    rs = (rs0, rs1)
    rd = (rd0, rd1)
    g = (g0, g1)
    sem = (sem0, sem1)

    zv = jnp.zeros((16,), f32)

    def zb(i, carry):
        for gg in range(8):
            g0[i, pl.ds(gg * 16, 16)] = zv
        return carry

    lax.fori_loop(0, CH4, zb, 0)

    def zc(i, carry):
        c = i * NTPC + sid

        @pl.when(c < NROWC)
        def _():
            pltpu.sync_copy(g0.at[pl.ds(0, ROWC)],
                            aggr.at[pl.ds(c * ROWC, ROWC)])
        return carry

    lax.fori_loop(0, (NROWC + NTPC - 1) // NTPC, zc, 0)
    plsc.subcore_barrier()

    def _base(k):
        c = k * NW + wid
        cc = jnp.minimum(c, NC4 - 1)
        return cc * CH4, cc, c

    def _fetch(q, rbase):
        pltpu.sync_copy(srcr.at[pl.ds(rbase, 1)], rs[q])
        pltpu.sync_copy(dstr.at[pl.ds(rbase, 1)], rd[q])
        pltpu.async_copy(h2_hbm.at[rs[q].at[0]], g[q], sem[q])

    base0, rbase0, _ = _base(0)
    _fetch(0, rbase0)

    def step(i, carry):
        for b in (0, 1):
            k = i * 2 + b
            q = 1 - b
            baseb, _, c = _base(k)
            _, rbasen, _ = _base(k + 1)
            _fetch(q, rbasen)
            pltpu.sync_copy(q3.at[pl.ds(baseb, CH4)], qbuf)
            pltpu.make_async_copy(h2_hbm.at[pl.ds(0, CH4)], g[b],
                                  sem[b]).wait()

            def rowop(r, carry2):
                for gg in range(8):
                    sl = pl.ds(gg * 16, 16)
                    g[b][r, sl] = jnp.maximum(g[b][r, sl] + qbuf[r, sl], 0.0)
                return carry2

            lax.fori_loop(0, CH4, rowop, 0)

            @pl.when(c < NC4)
            def _():
                pltpu.sync_copy(g[b], aggr.at[rd[b].at[0]], add=True)
        return carry

    lax.fori_loop(0, IT4 // 2, step, 0)
    pltpu.make_async_copy(h2_hbm.at[pl.ds(0, CH4)], g[0], sem[0]).wait()
    plsc.subcore_barrier()

    def wb(i, carry):
        c = i * NTPC + sid

        @pl.when(c < NROWC)
        def _():
            base = c * ROWC
            hsl = g0.at[pl.ds(0, ROWC)]
            pltpu.sync_copy(aggr.at[pl.ds(base, ROWC)], hsl)

            @pl.when(cid == 0)
            def _():
                pltpu.sync_copy(h2_hbm.at[pl.ds(base, ROWC)],
                                g1.at[pl.ds(0, ROWC)])

                def row(r, carry2):
                    for gg in range(8):
                        sl = pl.ds(gg * 16, 16)
                        g0[r, sl] = g0[r, sl] + g1[r, sl]
                    return carry2

                lax.fori_loop(0, ROWC, row, 0)
            pltpu.sync_copy(hsl, p3.at[pl.ds(cid * N + base, ROWC)])
        return carry

    lax.fori_loop(0, (NROWC + NTPC - 1) // NTPC, wb, 0)


def _k4(srcr, dstr, q3, h2):
    return pl.kernel(
        _k4_body,
        out_type=jax.ShapeDtypeStruct((2 * N, H), f32),
        mesh=_mesh(),
        scratch_types=[
            pltpu.VMEM((1, 128), i32),
            pltpu.VMEM((1, 128), i32),
            pltpu.VMEM((1, 128), i32),
            pltpu.VMEM((1, 128), i32),
            pltpu.VMEM((CH4, H), f32),
            pltpu.VMEM((CH4, H), f32),
            pltpu.VMEM((CH4, H), f32),
            pltpu.VMEM_SHARED((N, H), f32),
            pltpu.SemaphoreType.DMA,
            pltpu.SemaphoreType.DMA,
        ],
    )(srcr, dstr, q3, h2)


BLK = 1000


def _mm_body(pa, pb, w, b, o):
    acc = pa[...] + pb[...]
    o[...] = jnp.dot(acc, w[...], preferred_element_type=f32) + b[...]


def _mm(p, w, b):
    return pl.pallas_call(
        _mm_body,
        grid=(N // BLK,),
        in_specs=[
            pl.BlockSpec((BLK, H), lambda i: (i, 0)),
            pl.BlockSpec((BLK, H), lambda i: (i + N // BLK, 0)),
            pl.BlockSpec((H, H), lambda i: (0, 0)),
            pl.BlockSpec((1, H), lambda i: (0, 0)),
        ],
        out_specs=pl.BlockSpec((BLK, H), lambda i: (i, 0)),
        out_shape=jax.ShapeDtypeStruct((N, H), f32),
    )(p, p, w, b)


def _k5_body(pa, pb, w3, b3, bt, wf1, bf1, wf2, bf2, out, pooled):
    i = pl.program_id(0)

    @pl.when(i == 0)
    def _():
        pooled[...] = jnp.full((G, H), -jnp.inf, f32)

    h3 = jnp.dot(pa[...] + pb[...], w3[...],
                 preferred_element_type=f32) + b3[...]
    b = bt[0, 0, :]
    bmin = jnp.min(b)
    bmax = jnp.max(b)
    bc = b[:, None]
    for g in range(G):
        @pl.when((bmin <= g) & (g <= bmax))
        def _(g=g):
            cand = jnp.where(bc == g, h3, -jnp.inf)
            m = jnp.max(cand, axis=0, keepdims=True)
            pooled[pl.ds(g, 1), :] = jnp.maximum(pooled[pl.ds(g, 1), :], m)

    @pl.when(i == pl.num_programs(0) - 1)
    def _():
        p = pooled[...]
        e1 = jnp.maximum(
            jnp.dot(p, wf1[...], preferred_element_type=f32) + bf1[...], 0.0)
        out[...] = jnp.dot(e1, wf2[...], preferred_element_type=f32) + bf2[...]


def _k5(p, w3, b3, bt, wf1, bf1, wf2, bf2):
    nb = N // BLK
    return pl.pallas_call(
        _k5_body,
        grid=(nb,),
        in_specs=[
            pl.BlockSpec((BLK, H), lambda i: (i, 0)),
            pl.BlockSpec((BLK, H), lambda i: (i + nb, 0)),
            pl.BlockSpec((H, H), lambda i: (0, 0)),
            pl.BlockSpec((1, H), lambda i: (0, 0)),
            pl.BlockSpec((1, 1, BLK), lambda i: (i, 0, 0)),
            pl.BlockSpec((H, H), lambda i: (0, 0)),
            pl.BlockSpec((1, H), lambda i: (0, 0)),
            pl.BlockSpec((H, 1), lambda i: (0, 0)),
            pl.BlockSpec((1, 1), lambda i: (0, 0)),
        ],
        out_specs=pl.BlockSpec((G, 1), lambda i: (0, 0)),
        out_shape=jax.ShapeDtypeStruct((G, 1), f32),
        scratch_shapes=[pltpu.VMEM((G, H), f32)],
    )(p, p, w3, b3, bt, wf1, bf1, wf2, bf2)


def kernel(x, edge_index, edge_attr, batch, opt_edge, We, be, Weo, beo,
           Wl1, bl1, Wn1, bn1, Wl2, bl2, Wn2, bn2, Wl3, bl3, Wn3, bn3,
           Wf1, bf1, Wf2, bf2):
    src = edge_index[0]
    dst = edge_index[1]
    srcr = src.reshape(E // 128, 128)
    dstr = dst.reshape(E // 128, 128)
    xf = x[:, 0]
    coef2 = jnp.stack([Wn1[0], bn1])

    ea1, q2, q3 = _t01(edge_attr, opt_edge, We, be, Weo, beo,
                       Wl1, bl1, Wl2, bl2, Wl3, bl3)
    s = _k1(srcr, dstr, ea1[:, 0], xf)
    p2 = _k2(srcr, dstr, q2, s, coef2)
    h2 = _mm(p2, Wn2, bn2.reshape(1, H))
    p3 = _k4(srcr, dstr, q3, h2)
    return _k5(p3, Wn3, bn3.reshape(1, H), batch.reshape(N // BLK, 1, BLK),
               Wf1, bf1.reshape(1, H), Wf2, bf2.reshape(1, 1))


# speedup vs baseline: 1.1502x; 1.1502x over previous
import functools

import jax
import jax.numpy as jnp
from jax import lax
from jax.experimental import pallas as pl
from jax.experimental.pallas import tpu as pltpu
from jax.experimental.pallas import tpu_sc as plsc

N, E, G, H = 10000, 320000, 64, 128
ROWC = 80
NROWC = N // ROWC
NTPC = 16
NW = 32
f32 = jnp.float32
i32 = jnp.int32

CH1 = 512
KR1 = CH1 // 128
NC1 = E // CH1
IT1 = 40
CH2 = 128
NC2 = E // CH2
IT2 = 80
CH4 = 128
NC4 = E // CH4
IT4 = 80


def _mesh():
    return plsc.VectorSubcoreMesh(core_axis_name="c", subcore_axis_name="s")


EBLK = 8000


def _t01_body(aref, oref, we, be_r, weo, beo_r, wl1, bl1r, wl2, bl2r,
              wl3, bl3r, ea1_o, q2_o, q3_o):
    a = aref[...]
    o = oref[...]
    eel = a * we[...] + be_r[...]
    eer = o * weo[...] + beo_r[...]
    ee = jnp.concatenate([eel, eer], axis=1)
    ea1_o[...] = jnp.dot(ee, wl1[...], preferred_element_type=f32) + bl1r[...]
    q2_o[...] = jnp.dot(ee, wl2[...], preferred_element_type=f32) + bl2r[...]
    q3_o[...] = jnp.dot(ee, wl3[...], preferred_element_type=f32) + bl3r[...]


def _t01(a2, o2, We, be, Weo, beo, Wl1, bl1, Wl2, bl2, Wl3, bl3):
    nb = E // EBLK
    return pl.pallas_call(
        _t01_body,
        grid=(nb,),
        in_specs=[
            pl.BlockSpec((EBLK, 1), lambda i: (i, 0)),
            pl.BlockSpec((EBLK, 1), lambda i: (i, 0)),
            pl.BlockSpec((1, 64), lambda i: (0, 0)),
            pl.BlockSpec((1, 64), lambda i: (0, 0)),
            pl.BlockSpec((1, 64), lambda i: (0, 0)),
            pl.BlockSpec((1, 64), lambda i: (0, 0)),
            pl.BlockSpec((H, 1), lambda i: (0, 0)),
            pl.BlockSpec((1, 1), lambda i: (0, 0)),
            pl.BlockSpec((H, H), lambda i: (0, 0)),
            pl.BlockSpec((1, H), lambda i: (0, 0)),
            pl.BlockSpec((H, H), lambda i: (0, 0)),
            pl.BlockSpec((1, H), lambda i: (0, 0)),
        ],
        out_specs=[
            pl.BlockSpec((EBLK, 1), lambda i: (i, 0)),
            pl.BlockSpec((EBLK, H), lambda i: (i, 0)),
            pl.BlockSpec((EBLK, H), lambda i: (i, 0)),
        ],
        out_shape=[
            jax.ShapeDtypeStruct((E, 1), f32),
            jax.ShapeDtypeStruct((E, H), f32),
            jax.ShapeDtypeStruct((E, H), f32),
        ],
    )(a2, o2, We, be.reshape(1, 64), Weo, beo.reshape(1, 64),
      Wl1, bl1.reshape(1, 1), Wl2, bl2.reshape(1, H), Wl3, bl3.reshape(1, H))


def _k1_body(srcr, dstr, ea1, xf, s_out,
             rs0, rs1, rd0, rd1, fb0, fb1, g0, g1, mbuf, aggr, sem0, sem1):
    cid = lax.axis_index("c")
    sid = lax.axis_index("s")
    rs = (rs0, rs1)
    rd = (rd0, rd1)
    fb = (fb0, fb1)
    g = (g0, g1)
    sem = (sem0, sem1)

    @pl.when(cid == 0)
    def _core0():
        def initb(i, carry):
            c = i * NTPC + sid

            @pl.when(c < NROWC)
            def _():
                pltpu.sync_copy(xf.at[pl.ds(c * ROWC, ROWC)],
                                mbuf.at[pl.ds(0, ROWC)])
                pltpu.sync_copy(mbuf.at[pl.ds(0, ROWC)],
                                aggr.at[pl.ds(c * ROWC, ROWC)])
            return carry

        lax.fori_loop(0, (NROWC + NTPC - 1) // NTPC, initb, 0)
        plsc.subcore_barrier()

        def _base(k):
            c = k * NTPC + sid
            cc = jnp.minimum(c, NC1 - 1)
            return cc * CH1, cc * KR1, c

        def _gather(q):
            for j in range(KR1):
                pltpu.async_copy(xf.at[rs[q].at[j]],
                                 g[q].at[pl.ds(j * 128, 128)], sem[q])

        base0, rbase0, _ = _base(0)
        pltpu.sync_copy(srcr.at[pl.ds(rbase0, KR1)], rs[0])
        pltpu.sync_copy(dstr.at[pl.ds(rbase0, KR1)], rd[0])
        pltpu.sync_copy(ea1.at[pl.ds(base0, CH1)], fb[0])
        _gather(0)

        def step(i, carry):
            for b in (0, 1):
                k = i * 2 + b
                q = 1 - b
                _, _, c = _base(k)
                basen, rbasen, _ = _base(k + 1)
                pltpu.sync_copy(srcr.at[pl.ds(rbasen, KR1)], rs[q])
                pltpu.sync_copy(dstr.at[pl.ds(rbasen, KR1)], rd[q])
                pltpu.sync_copy(ea1.at[pl.ds(basen, CH1)], fb[q])
                _gather(q)
                pltpu.make_async_copy(xf.at[pl.ds(0, CH1)], g[b],
                                      sem[b]).wait()
                for gg in range(CH1 // 16):
                    sl = pl.ds(gg * 16, 16)
                    mbuf[sl] = jnp.maximum(g[b][sl] + fb[b][sl], 0.0)

                @pl.when(c < NC1)
                def _():
                    for j in range(KR1):
                        pltpu.sync_copy(mbuf.at[pl.ds(j * 128, 128)],
                                        aggr.at[rd[b].at[j]], add=True)
            return carry

        lax.fori_loop(0, IT1 // 2, step, 0)
        pltpu.make_async_copy(xf.at[pl.ds(0, CH1)], g[0], sem[0]).wait()
        plsc.subcore_barrier()

        def wb(i, carry):
            c = i * NTPC + sid

            @pl.when(c < NROWC)
            def _():
                pltpu.sync_copy(aggr.at[pl.ds(c * ROWC, ROWC)],
                                mbuf.at[pl.ds(0, ROWC)])
                pltpu.sync_copy(mbuf.at[pl.ds(0, ROWC)],
                                s_out.at[pl.ds(c * ROWC, ROWC)])
            return carry

        lax.fori_loop(0, (NROWC + NTPC - 1) // NTPC, wb, 0)


def _k1(srcr, dstr, ea1, xf):
    return pl.kernel(
        _k1_body,
        out_type=jax.ShapeDtypeStruct((N,), f32),
        mesh=_mesh(),
        scratch_types=[
            pltpu.VMEM((KR1, 128), i32),
            pltpu.VMEM((KR1, 128), i32),
            pltpu.VMEM((KR1, 128), i32),
            pltpu.VMEM((KR1, 128), i32),
            pltpu.VMEM((CH1,), f32),
            pltpu.VMEM((CH1,), f32),
            pltpu.VMEM((CH1,), f32),
            pltpu.VMEM((CH1,), f32),
            pltpu.VMEM((CH1,), f32),
            pltpu.VMEM_SHARED((N,), f32),
            pltpu.SemaphoreType.DMA,
            pltpu.SemaphoreType.DMA,
        ],
    )(srcr, dstr, ea1, xf)


def _k2_body(srcr, dstr, q2, s_hbm, coef, p2,
             rs0, rs1, rd0, rd1, qb0, qb1, g0, g1, cbuf, aggr, sem0, sem1):
    cid = lax.axis_index("c")
    sid = lax.axis_index("s")
    wid = cid * NTPC + sid
    rs = (rs0, rs1)
    rd = (rd0, rd1)
    qb = (qb0, qb1)
    g = (g0, g1)
    sem = (sem0, sem1)

    pltpu.sync_copy(coef, cbuf)
    w1s = [cbuf[0, pl.ds(gg * 16, 16)] for gg in range(8)]
    bns = [cbuf[1, pl.ds(gg * 16, 16)] for gg in range(8)]

    zv = jnp.zeros((16,), f32)

    def zb(i, carry):
        for gg in range(8):
            qb0[i, pl.ds(gg * 16, 16)] = zv
        return carry

    lax.fori_loop(0, CH2, zb, 0)

    def zc(i, carry):
        c = i * NTPC + sid

        @pl.when(c < NROWC)
        def _():
            pltpu.sync_copy(qb0.at[pl.ds(0, ROWC)],
                            aggr.at[pl.ds(c * ROWC, ROWC)])
        return carry

    lax.fori_loop(0, (NROWC + NTPC - 1) // NTPC, zc, 0)
    plsc.subcore_barrier()

    def _base(k):
        c = k * NW + wid
        cc = jnp.minimum(c, NC2 - 1)
        return cc * CH2, cc, c

    def _fetch(q, base, rbase):
        pltpu.sync_copy(srcr.at[pl.ds(rbase, 1)], rs[q])
        pltpu.sync_copy(dstr.at[pl.ds(rbase, 1)], rd[q])
        pltpu.async_copy(q2.at[pl.ds(base, CH2)], qb[q], sem[q])
        pltpu.async_copy(s_hbm.at[rs[q].at[0]], g[q], sem[q])

    base0, rbase0, _ = _base(0)
    _fetch(0, base0, rbase0)

    def step(i, carry):
        for b in (0, 1):
            k = i * 2 + b
            q = 1 - b
            _, _, c = _base(k)
            basen, rbasen, _ = _base(k + 1)
            _fetch(q, basen, rbasen)
            pltpu.make_async_copy(q2.at[pl.ds(0, CH2)], qb[b], sem[b]).wait()
            pltpu.make_async_copy(s_hbm.at[pl.ds(0, CH2)], g[b],
                                  sem[b]).wait()

            def edgegrp(qq, carry2):
                qbase = qq * 16
                sv = g[b][pl.ds(qbase, 16)]
                for e in range(16):
                    se = sv[e]
                    for gg in range(8):
                        sl = pl.ds(gg * 16, 16)
                        qb[b][qbase + e, sl] = jnp.maximum(
                            (se * w1s[gg] + bns[gg]) + qb[b][qbase + e, sl],
                            0.0)
                return carry2

            lax.fori_loop(0, CH2 // 16, edgegrp, 0)

            @pl.when(c < NC2)
            def _():
                pltpu.sync_copy(qb[b], aggr.at[rd[b].at[0]], add=True)
        return carry

    lax.fori_loop(0, IT2 // 2, step, 0)
    pltpu.make_async_copy(q2.at[pl.ds(0, CH2)], qb[0], sem[0]).wait()
    pltpu.make_async_copy(s_hbm.at[pl.ds(0, CH2)], g[0], sem[0]).wait()
    plsc.subcore_barrier()

    def wb(i, carry):
        c = i * NTPC + sid

        @pl.when(c < NROWC)
        def _():
            base = c * ROWC
            msl = qb0.at[pl.ds(0, ROWC)]
            pltpu.sync_copy(aggr.at[pl.ds(base, ROWC)], msl)

            @pl.when(cid == 0)
            def _():
                pltpu.sync_copy(s_hbm.at[pl.ds(base, ROWC)],
                                g0.at[pl.ds(0, ROWC)])

                def rowgrp(qq, carry2):
                    qbase = qq * 16
                    sv = g0[pl.ds(qbase, 16)]
                    for e in range(16):
                        se = sv[e]
                        for gg in range(8):
                            sl = pl.ds(gg * 16, 16)
                            qb0[qbase + e, sl] = qb0[qbase + e, sl] \
                                + se * w1s[gg] + bns[gg]
                    return carry2

                lax.fori_loop(0, ROWC // 16, rowgrp, 0)
            pltpu.sync_copy(msl, p2.at[pl.ds(cid * N + base, ROWC)])
        return carry

    lax.fori_loop(0, (NROWC + NTPC - 1) // NTPC, wb, 0)


def _k2(srcr, dstr, q2, s, coef):
    return pl.kernel(
        _k2_body,
        out_type=jax.ShapeDtypeStruct((2 * N, H), f32),
        mesh=_mesh(),
        scratch_types=[
            pltpu.VMEM((1, 128), i32),
            pltpu.VMEM((1, 128), i32),
            pltpu.VMEM((1, 128), i32),
            pltpu.VMEM((1, 128), i32),
            pltpu.VMEM((CH2, H), f32),
            pltpu.VMEM((CH2, H), f32),
            pltpu.VMEM((CH2,), f32),
            pltpu.VMEM((CH2,), f32),
            pltpu.VMEM((2, H), f32),
            pltpu.VMEM_SHARED((N, H), f32),
            pltpu.SemaphoreType.DMA,
            pltpu.SemaphoreType.DMA,
        ],
    )(srcr, dstr, q2, s, coef)


def _k4_body(srcr, dstr, q3, h2_hbm, p3,
             rs0, rs1, rd0, rd1, qbuf, g0, g1, aggr, sem0, sem1):
    cid = lax.axis_index("c")
    sid = lax.axis_index("s")
    wid = cid * NTPC + sid
    rs = (rs0, rs1)
    rd = (rd0, rd1)
    g = (g0, g1)
    sem = (sem0, sem1)

    zv = jnp.zeros((16,), f32)

    def zb(i, carry):
        for gg in range(8):
            g0[i, pl.ds(gg * 16, 16)] = zv
        return carry

    lax.fori_loop(0, CH4, zb, 0)

    def zc(i, carry):
        c = i * NTPC + sid

        @pl.when(c < NROWC)
        def _():
            pltpu.sync_copy(g0.at[pl.ds(0, ROWC)],
                            aggr.at[pl.ds(c * ROWC, ROWC)])
        return carry

    lax.fori_loop(0, (NROWC + NTPC - 1) // NTPC, zc, 0)
    plsc.subcore_barrier()

    def _base(k):
        c = k * NW + wid
        cc = jnp.minimum(c, NC4 - 1)
        return cc * CH4, cc, c

    def _fetch(q, rbase):
        pltpu.sync_copy(srcr.at[pl.ds(rbase, 1)], rs[q])
        pltpu.sync_copy(dstr.at[pl.ds(rbase, 1)], rd[q])
        pltpu.async_copy(h2_hbm.at[rs[q].at[0]], g[q], sem[q])

    base0, rbase0, _ = _base(0)
    _fetch(0, rbase0)

    def step(i, carry):
        for b in (0, 1):
            k = i * 2 + b
            q = 1 - b
            baseb, _, c = _base(k)
            _, rbasen, _ = _base(k + 1)
            _fetch(q, rbasen)
            pltpu.sync_copy(q3.at[pl.ds(baseb, CH4)], qbuf)
            pltpu.make_async_copy(h2_hbm.at[pl.ds(0, CH4)], g[b],
                                  sem[b]).wait()

            def rowop(r, carry2):
                for gg in range(8):
                    sl = pl.ds(gg * 16, 16)
                    g[b][r, sl] = jnp.maximum(g[b][r, sl] + qbuf[r, sl], 0.0)
                return carry2

            lax.fori_loop(0, CH4, rowop, 0)

            @pl.when(c < NC4)
            def _():
                pltpu.sync_copy(g[b], aggr.at[rd[b].at[0]], add=True)
        return carry

    lax.fori_loop(0, IT4 // 2, step, 0)
    pltpu.make_async_copy(h2_hbm.at[pl.ds(0, CH4)], g[0], sem[0]).wait()
    plsc.subcore_barrier()

    def wb(i, carry):
        c = i * NTPC + sid

        @pl.when(c < NROWC)
        def _():
            base = c * ROWC
            hsl = g0.at[pl.ds(0, ROWC)]
            pltpu.sync_copy(aggr.at[pl.ds(base, ROWC)], hsl)

            @pl.when(cid == 0)
            def _():
                pltpu.sync_copy(h2_hbm.at[pl.ds(base, ROWC)],
                                g1.at[pl.ds(0, ROWC)])

                def row(r, carry2):
                    for gg in range(8):
                        sl = pl.ds(gg * 16, 16)
                        g0[r, sl] = g0[r, sl] + g1[r, sl]
                    return carry2

                lax.fori_loop(0, ROWC, row, 0)
            pltpu.sync_copy(hsl, p3.at[pl.ds(cid * N + base, ROWC)])
        return carry

    lax.fori_loop(0, (NROWC + NTPC - 1) // NTPC, wb, 0)


def _k4(srcr, dstr, q3, h2):
    return pl.kernel(
        _k4_body,
        out_type=jax.ShapeDtypeStruct((2 * N, H), f32),
        mesh=_mesh(),
        scratch_types=[
            pltpu.VMEM((1, 128), i32),
            pltpu.VMEM((1, 128), i32),
            pltpu.VMEM((1, 128), i32),
            pltpu.VMEM((1, 128), i32),
            pltpu.VMEM((CH4, H), f32),
            pltpu.VMEM((CH4, H), f32),
            pltpu.VMEM((CH4, H), f32),
            pltpu.VMEM_SHARED((N, H), f32),
            pltpu.SemaphoreType.DMA,
            pltpu.SemaphoreType.DMA,
        ],
    )(srcr, dstr, q3, h2)


BLK = 1000


def _mm_body(pa, pb, w, b, o):
    acc = pa[...] + pb[...]
    o[...] = jnp.dot(acc, w[...], preferred_element_type=f32) + b[...]


def _mm(p, w, b):
    return pl.pallas_call(
        _mm_body,
        grid=(N // BLK,),
        in_specs=[
            pl.BlockSpec((BLK, H), lambda i: (i, 0)),
            pl.BlockSpec((BLK, H), lambda i: (i + N // BLK, 0)),
            pl.BlockSpec((H, H), lambda i: (0, 0)),
            pl.BlockSpec((1, H), lambda i: (0, 0)),
        ],
        out_specs=pl.BlockSpec((BLK, H), lambda i: (i, 0)),
        out_shape=jax.ShapeDtypeStruct((N, H), f32),
    )(p, p, w, b)


def _k5_body(pa, pb, w3, b3, bt, wf1, bf1, wf2, bf2, out, pooled):
    i = pl.program_id(0)

    @pl.when(i == 0)
    def _():
        pooled[...] = jnp.full((G, H), -jnp.inf, f32)

    h3 = jnp.dot(pa[...] + pb[...], w3[...],
                 preferred_element_type=f32) + b3[...]
    b = bt[0, 0, :]
    bmin = jnp.min(b)
    bmax = jnp.max(b)
    bc = b[:, None]
    for g in range(G):
        @pl.when((bmin <= g) & (g <= bmax))
        def _(g=g):
            cand = jnp.where(bc == g, h3, -jnp.inf)
            m = jnp.max(cand, axis=0, keepdims=True)
            pooled[pl.ds(g, 1), :] = jnp.maximum(pooled[pl.ds(g, 1), :], m)

    @pl.when(i == pl.num_programs(0) - 1)
    def _():
        p = pooled[...]
        e1 = jnp.maximum(
            jnp.dot(p, wf1[...], preferred_element_type=f32) + bf1[...], 0.0)
        out[...] = jnp.dot(e1, wf2[...], preferred_element_type=f32) + bf2[...]


def _k5(p, w3, b3, bt, wf1, bf1, wf2, bf2):
    nb = N // BLK
    return pl.pallas_call(
        _k5_body,
        grid=(nb,),
        in_specs=[
            pl.BlockSpec((BLK, H), lambda i: (i, 0)),
            pl.BlockSpec((BLK, H), lambda i: (i + nb, 0)),
            pl.BlockSpec((H, H), lambda i: (0, 0)),
            pl.BlockSpec((1, H), lambda i: (0, 0)),
            pl.BlockSpec((1, 1, BLK), lambda i: (i, 0, 0)),
            pl.BlockSpec((H, H), lambda i: (0, 0)),
            pl.BlockSpec((1, H), lambda i: (0, 0)),
            pl.BlockSpec((H, 1), lambda i: (0, 0)),
            pl.BlockSpec((1, 1), lambda i: (0, 0)),
        ],
        out_specs=pl.BlockSpec((G, 1), lambda i: (0, 0)),
        out_shape=jax.ShapeDtypeStruct((G, 1), f32),
        scratch_shapes=[pltpu.VMEM((G, H), f32)],
    )(p, p, w3, b3, bt, wf1, bf1, wf2, bf2)


def kernel(x, edge_index, edge_attr, batch, opt_edge, We, be, Weo, beo,
           Wl1, bl1, Wn1, bn1, Wl2, bl2, Wn2, bn2, Wl3, bl3, Wn3, bn3,
           Wf1, bf1, Wf2, bf2):
    src = edge_index[0]
    dst = edge_index[1]
    srcr = src.reshape(E // 128, 128)
    dstr = dst.reshape(E // 128, 128)
    xf = x[:, 0]
    coef2 = jnp.stack([Wn1[0], bn1])

    ea1, q2, q3 = _t01(edge_attr, opt_edge, We, be, Weo, beo,
                       Wl1, bl1, Wl2, bl2, Wl3, bl3)
    s = _k1(srcr, dstr, ea1.reshape(E), xf)
    p2 = _k2(srcr, dstr, q2, s, coef2)
    h2 = _mm(p2, Wn2, bn2.reshape(1, H))
    p3 = _k4(srcr, dstr, q3, h2)
    return _k5(p3, Wn3, bn3.reshape(1, H), batch.reshape(N // BLK, 1, BLK),
               Wf1, bf1.reshape(1, H), Wf2, bf2.reshape(1, 1))
